# baseline (speedup 1.0000x reference)
import jax
import jax.numpy as jnp
from jax.experimental import pallas as pl

_K = 512
_D = 64
_HW = 1024


def _vq_kernel(x_ref, cb_ref, enc_ref, q_ref):
    xb = x_ref[...].reshape(_D, _HW)
    cb = cb_ref[...]
    scores = jax.lax.dot_general(
        cb, xb, (((1,), (0,)), ((), ())), preferred_element_type=jnp.float32)
    c_sq = jnp.sum(cb * cb, axis=1, keepdims=True)
    d2 = c_sq - 2.0 * scores
    idx = jnp.argmin(d2, axis=0)
    enc = (jax.lax.broadcasted_iota(jnp.int32, (_HW, _K), 1)
           == idx[:, None]).astype(jnp.float32)
    enc_ref[...] = enc
    q = jax.lax.dot_general(
        cb, enc, (((0,), (1,)), ((), ())), preferred_element_type=jnp.float32)
    q_ref[...] = q.reshape(1, _D, 32, 32)


def kernel(x, codebook):
    b = x.shape[0]
    n = b * _HW
    enc, q = pl.pallas_call(
        _vq_kernel,
        grid=(b,),
        in_specs=[
            pl.BlockSpec((1, _D, 32, 32), lambda i: (i, 0, 0, 0)),
            pl.BlockSpec((_K, _D), lambda i: (0, 0)),
        ],
        out_specs=[
            pl.BlockSpec((_HW, _K), lambda i: (i, 0)),
            pl.BlockSpec((1, _D, 32, 32), lambda i: (i, 0, 0, 0)),
        ],
        out_shape=[
            jax.ShapeDtypeStruct((n, _K), jnp.float32),
            jax.ShapeDtypeStruct(x.shape, jnp.float32),
        ],
    )(x, codebook)
    return (enc, q)


# speedup vs baseline: 1.6081x; 1.6081x over previous
import jax
import jax.numpy as jnp
from jax.experimental import pallas as pl

_K = 512
_D = 64
_HW = 1024


def _vq_kernel(x_ref, cb_ref, enc_ref, q_ref):
    xb = x_ref[0]
    cb = cb_ref[...]
    scores = jax.lax.dot_general(
        cb, xb, (((1,), (0,)), ((), ())), preferred_element_type=jnp.float32)
    c_sq = jnp.sum(cb * cb, axis=1, keepdims=True)
    d2 = c_sq - 2.0 * scores
    idx = jnp.argmin(d2, axis=0)
    enc = (jax.lax.broadcasted_iota(jnp.int32, (_HW, _K), 1)
           == idx[:, None]).astype(jnp.float32)
    enc_ref[...] = enc
    q = jax.lax.dot_general(
        cb, enc, (((0,), (1,)), ((), ())), preferred_element_type=jnp.float32)
    q_ref[...] = q[None]


def kernel(x, codebook):
    b = x.shape[0]
    n = b * _HW
    x3 = x.reshape(b, _D, _HW)
    enc, q = pl.pallas_call(
        _vq_kernel,
        grid=(b,),
        in_specs=[
            pl.BlockSpec((1, _D, _HW), lambda i: (i, 0, 0)),
            pl.BlockSpec((_K, _D), lambda i: (0, 0)),
        ],
        out_specs=[
            pl.BlockSpec((_HW, _K), lambda i: (i, 0)),
            pl.BlockSpec((1, _D, _HW), lambda i: (i, 0, 0)),
        ],
        out_shape=[
            jax.ShapeDtypeStruct((n, _K), jnp.float32),
            jax.ShapeDtypeStruct((b, _D, _HW), jnp.float32),
        ],
    )(x3, codebook)
    return (enc, q.reshape(x.shape))
